# Initial kernel scaffold; baseline (speedup 1.0000x reference)
import jax
import jax.numpy as jnp
from jax.experimental import pallas as pl


def kernel(x, edge_index, W1, b1, g1, be1, Wg, bg, g2, be2):
    raise NotImplementedError("write your pallas kernel here")


# speedup vs baseline: 19.0482x; 19.0482x over previous
import functools

import jax
import jax.numpy as jnp
from jax import lax
from jax.experimental import pallas as pl
from jax.experimental.pallas import tpu as pltpu
from jax.experimental.pallas import tpu_sc as plsc

N = 10000
D = 128
EPS = 1e-5

NC = 2
NS = 16
NW = NC * NS
CH = 128

NPAD = 10112
ROWS_PER_TILE = NPAD // NS


def _make_deg_kernel(k_chunks):
  mesh = plsc.VectorSubcoreMesh(core_axis_name="c", subcore_axis_name="s")

  @functools.partial(
      pl.kernel,
      out_type=jax.ShapeDtypeStruct((NC * NPAD,), jnp.float32),
      mesh=mesh,
      scratch_types=[
          pltpu.VMEM((k_chunks, CH), jnp.int32),
          pltpu.VMEM((CH,), jnp.float32),
          pltpu.VMEM((ROWS_PER_TILE,), jnp.float32),
          pltpu.VMEM_SHARED((NPAD,), jnp.float32),
      ],
  )
  def deg_kernel(col_hbm, zero_hbm, deg_hbm, col_v, ones_v, tmp_v, deg_sh):
    cid = lax.axis_index("c")
    sid = lax.axis_index("s")
    wid = sid * NC + cid
    pltpu.sync_copy(col_hbm.at[wid], col_v)
    for i in range(CH // 16):
      ones_v[pl.ds(i * 16, 16)] = jnp.ones((16,), jnp.float32)
    base = sid * ROWS_PER_TILE
    pltpu.sync_copy(zero_hbm, tmp_v)
    pltpu.sync_copy(tmp_v, deg_sh.at[pl.ds(base, ROWS_PER_TILE)])
    plsc.subcore_barrier()

    @pl.loop(0, k_chunks)
    def _(j):
      pltpu.sync_copy(ones_v, deg_sh.at[col_v.at[j]], add=True)

    plsc.subcore_barrier()
    pltpu.sync_copy(deg_sh.at[pl.ds(base, ROWS_PER_TILE)], tmp_v)
    pltpu.sync_copy(tmp_v, deg_hbm.at[pl.ds(cid * NPAD + base, ROWS_PER_TILE)])

  return deg_kernel


def _make_scatter_kernel(k_chunks):
  mesh = plsc.VectorSubcoreMesh(core_axis_name="c", subcore_axis_name="s")

  @functools.partial(
      pl.kernel,
      out_type=jax.ShapeDtypeStruct((NC, NPAD, D), jnp.float32),
      mesh=mesh,
      scratch_types=[
          pltpu.VMEM((k_chunks, CH), jnp.int32),
          pltpu.VMEM((k_chunks, CH), jnp.int32),
          pltpu.VMEM((CH, D), jnp.float32),
          pltpu.VMEM_SHARED((NPAD, D), jnp.float32),
          pltpu.SemaphoreType.DMA,
      ],
  )
  def scat_kernel(y_hbm, row_hbm, col_hbm, zero_hbm, out_hbm,
                  row_v, col_v, buf_v, acc_sh, sem):
    cid = lax.axis_index("c")
    sid = lax.axis_index("s")
    wid = sid * NC + cid
    pltpu.sync_copy(row_hbm.at[wid], row_v)
    pltpu.sync_copy(col_hbm.at[wid], col_v)
    base = sid * ROWS_PER_TILE
    nfull = ROWS_PER_TILE // CH
    tail = ROWS_PER_TILE - nfull * CH
    pltpu.sync_copy(zero_hbm, buf_v)

    @pl.loop(0, nfull)
    def _(j):
      pltpu.sync_copy(buf_v, acc_sh.at[pl.ds(base + j * CH, CH)])

    pltpu.sync_copy(buf_v.at[pl.ds(0, tail)],
                    acc_sh.at[pl.ds(base + nfull * CH, tail)])
    plsc.subcore_barrier()

    @pl.loop(0, k_chunks)
    def _(j):
      pltpu.async_copy(y_hbm.at[row_v.at[j]], buf_v, sem).wait()
      pltpu.sync_copy(buf_v, acc_sh.at[col_v.at[j]], add=True)

    plsc.subcore_barrier()

    @pl.loop(0, nfull)
    def _(j):
      pltpu.sync_copy(acc_sh.at[pl.ds(base + j * CH, CH)], buf_v)
      pltpu.sync_copy(buf_v, out_hbm.at[cid, pl.ds(base + j * CH, CH)])

    pltpu.sync_copy(acc_sh.at[pl.ds(base + nfull * CH, tail)],
                    buf_v.at[pl.ds(0, tail)])
    pltpu.sync_copy(buf_v.at[pl.ds(0, tail)],
                    out_hbm.at[cid, pl.ds(base + nfull * CH, tail)])

  return scat_kernel


def _mlp_body(x_ref, w1_ref, b1_ref, g1_ref, be1_ref, wg_ref, xh_ref):
  h = jnp.dot(x_ref[...], w1_ref[...], preferred_element_type=jnp.float32)
  h = h + b1_ref[...]
  mu = jnp.mean(h, axis=0, keepdims=True)
  c = h - mu
  var = jnp.mean(c * c, axis=0, keepdims=True)
  h = g1_ref[...] * c * lax.rsqrt(var + EPS) + be1_ref[...]
  h = jnp.maximum(h, 0.0)
  xh_ref[...] = jnp.dot(h, wg_ref[...], preferred_element_type=jnp.float32)


def _scale_body(xh_ref, p_ref, y_ref, dinv_ref):
  deg = p_ref[0] + p_ref[1] + 1.0
  dinv = lax.rsqrt(deg)
  dinv_ref[...] = dinv
  y_ref[pl.ds(0, N), :] = xh_ref[...] * dinv[0:N]
  y_ref[pl.ds(N, NPAD - N), :] = jnp.zeros((NPAD - N, D), jnp.float32)


def _final_body(s_ref, y_ref, dinv_ref, bg_ref, g2_ref, be2_ref, out_ref):
  s = s_ref[0, pl.ds(0, N), :] + s_ref[1, pl.ds(0, N), :] + y_ref[pl.ds(0, N), :]
  pre = dinv_ref[pl.ds(0, N), :] * s + bg_ref[...]
  mu = jnp.mean(pre, axis=0, keepdims=True)
  c = pre - mu
  var = jnp.mean(c * c, axis=0, keepdims=True)
  out_ref[...] = jnp.maximum(
      g2_ref[...] * c * lax.rsqrt(var + EPS) + be2_ref[...], 0.0)


def kernel(x, edge_index, W1, b1, g1, be1, Wg, bg, g2, be2):
  e = edge_index.shape[1]
  k_chunks = -(-e // (NW * CH))
  epad = NW * k_chunks * CH
  pad = epad - e

  row = jnp.concatenate(
      [edge_index[0], jnp.full((pad,), N, jnp.int32)]).reshape(NW, k_chunks, CH)
  col = jnp.concatenate(
      [edge_index[1], jnp.full((pad,), N, jnp.int32)]).reshape(NW, k_chunks, CH)
  zeros1 = jnp.zeros((ROWS_PER_TILE,), jnp.float32)
  zeros2 = jnp.zeros((CH, D), jnp.float32)

  deg_parts = _make_deg_kernel(k_chunks)(col, zeros1)

  xh = pl.pallas_call(
      _mlp_body,
      out_shape=jax.ShapeDtypeStruct((N, D), jnp.float32),
  )(x, W1, b1.reshape(1, D), g1.reshape(1, D), be1.reshape(1, D), Wg)

  y_pad, dinv = pl.pallas_call(
      _scale_body,
      out_shape=(jax.ShapeDtypeStruct((NPAD, D), jnp.float32),
                 jax.ShapeDtypeStruct((NPAD, 1), jnp.float32)),
  )(xh, deg_parts.reshape(NC, NPAD, 1))

  s_parts = _make_scatter_kernel(k_chunks)(y_pad, row, col, zeros2)

  out = pl.pallas_call(
      _final_body,
      out_shape=jax.ShapeDtypeStruct((N, D), jnp.float32),
  )(s_parts, y_pad, dinv, bg.reshape(1, D), g2.reshape(1, D), be2.reshape(1, D))
  return out
